# baseline (speedup 1.0000x reference)
import functools

import jax
import jax.numpy as jnp
from jax import lax
from jax.experimental import pallas as pl
from jax.experimental.pallas import tpu as pltpu
from jax.experimental.pallas import tpu_sc as plsc

_NC = 2
_NS = 16
_NW = _NC * _NS
_SPAD = 64


@functools.lru_cache(maxsize=None)
def _make_gather(V, D, B, S):
  bat_per_w = B // _NW
  npair = bat_per_w // 2
  chunk = 2 * _SPAD
  nbuf = 4
  mesh = plsc.VectorSubcoreMesh(core_axis_name="c", subcore_axis_name="s")

  @functools.partial(
      pl.kernel,
      mesh=mesh,
      out_type=jax.ShapeDtypeStruct((B, S, D), jnp.float32),
      scratch_types=[
          pltpu.VMEM((npair, chunk), jnp.int32),
          [pltpu.VMEM((chunk, D), jnp.float32) for _ in range(nbuf)],
          [pltpu.SemaphoreType.DMA for _ in range(nbuf)],
          [pltpu.SemaphoreType.DMA for _ in range(nbuf)],
      ],
      compiler_params=pltpu.CompilerParams(use_tc_tiling_on_sc=True),
  )
  def gather(table_hbm, idx_hbm, out_hbm, idx_v, bufs, gsems, wsems):
    wid = lax.axis_index("s") * _NC + lax.axis_index("c")
    bb = wid * bat_per_w
    pltpu.sync_copy(idx_hbm.at[wid], idx_v)

    def g(j, b):
      return pltpu.make_async_copy(table_hbm.at[idx_v.at[j]], bufs[b],
                                   gsems[b])

    def wr(j, b, half):
      return pltpu.make_async_copy(
          bufs[b].at[pl.ds(half * _SPAD, S)], out_hbm.at[bb + 2 * j + half],
          wsems[b])

    pf = 2
    for b in range(pf):
      g(b, b).start()

    def body(i, carry):
      for b in range(nbuf):
        j = nbuf * i + b
        g(j, b).wait()
        wr(j, b, 0).start()
        wr(j, b, 1).start()
        jp = j + pf
        bp = (b + pf) % nbuf

        @pl.when(jnp.logical_and(jp < npair, j >= nbuf - pf))
        def _():
          wr(jp, bp, 0).wait()
          wr(jp, bp, 1).wait()

        @pl.when(jp < npair)
        def _():
          g(jp, bp).start()
      return carry

    lax.fori_loop(0, npair // nbuf, body, 0)
    for b in range(nbuf):
      wr(0, b, 0).wait()
      wr(0, b, 1).wait()

  return gather


def _linear_tc(x, w):
  def mm(x_ref, w_ref, o_ref):
    o_ref[...] = lax.dot_general(
        x_ref[...], w_ref[...], (((1,), (1,)), ((), ())),
        preferred_element_type=jnp.float32)

  return pl.pallas_call(
      mm,
      out_shape=jax.ShapeDtypeStruct((x.shape[0], w.shape[0]), jnp.float32),
  )(x, w)


def kernel(token_ids, dense_feat, embedding_weight, linear_weight):
  B, S = token_ids.shape
  V, D = embedding_weight.shape
  idx = jnp.pad(token_ids.astype(jnp.int32), ((0, 0), (0, _SPAD - S)))
  idx2 = idx.reshape(_NW, B // (2 * _NW), 2 * _SPAD)
  emb_out = _make_gather(V, D, B, S)(embedding_weight, idx2)
  lin_out = _linear_tc(dense_feat.astype(jnp.float32),
                       linear_weight.astype(jnp.float32))
  return (emb_out, lin_out)


# speedup vs baseline: 14.7317x; 14.7317x over previous
import functools

import jax
import jax.numpy as jnp
from jax import lax
from jax.experimental import pallas as pl
from jax.experimental.pallas import tpu as pltpu
from jax.experimental.pallas import tpu_sc as plsc

_NC = 2
_NS = 16
_NW = _NC * _NS
_SPAD = 56


@functools.lru_cache(maxsize=None)
def _make_gather(V, D, B, S):
  bat_per_w = B // _NW
  npair = bat_per_w // 2
  chunk = 2 * _SPAD
  nbuf = 4
  mesh = plsc.VectorSubcoreMesh(core_axis_name="c", subcore_axis_name="s")

  @functools.partial(
      pl.kernel,
      mesh=mesh,
      out_type=jax.ShapeDtypeStruct((B, S, D), jnp.float32),
      scratch_types=[
          pltpu.VMEM((npair, chunk), jnp.int32),
          [pltpu.VMEM((chunk, D), jnp.float32) for _ in range(nbuf)],
          [pltpu.SemaphoreType.DMA for _ in range(nbuf)],
          [pltpu.SemaphoreType.DMA for _ in range(nbuf)],
      ],
      compiler_params=pltpu.CompilerParams(use_tc_tiling_on_sc=True),
  )
  def gather(table_hbm, idx_hbm, out_hbm, idx_v, bufs, gsems, wsems):
    wid = lax.axis_index("s") * _NC + lax.axis_index("c")
    bb = wid * bat_per_w
    pltpu.sync_copy(idx_hbm.at[wid], idx_v)

    def g(j, b):
      return pltpu.make_async_copy(table_hbm.at[idx_v.at[j]], bufs[b],
                                   gsems[b])

    def wr(j, b, half):
      return pltpu.make_async_copy(
          bufs[b].at[pl.ds(half * _SPAD, S)], out_hbm.at[bb + 2 * j + half],
          wsems[b])

    pf = 2
    for b in range(pf):
      g(b, b).start()

    def body(i, carry):
      for b in range(nbuf):
        j = nbuf * i + b
        g(j, b).wait()
        wr(j, b, 0).start()
        wr(j, b, 1).start()
        jp = j + pf
        bp = (b + pf) % nbuf

        @pl.when(jnp.logical_and(jp < npair, j >= nbuf - pf))
        def _():
          wr(jp, bp, 0).wait()
          wr(jp, bp, 1).wait()

        @pl.when(jp < npair)
        def _():
          g(jp, bp).start()
      return carry

    lax.fori_loop(0, npair // nbuf, body, 0)
    for b in range(nbuf):
      wr(0, b, 0).wait()
      wr(0, b, 1).wait()

  return gather


def _linear_tc(x, w):
  def mm(x_ref, w_ref, o_ref):
    o_ref[...] = lax.dot_general(
        x_ref[...], w_ref[...], (((1,), (1,)), ((), ())),
        preferred_element_type=jnp.float32)

  return pl.pallas_call(
      mm,
      out_shape=jax.ShapeDtypeStruct((x.shape[0], w.shape[0]), jnp.float32),
  )(x, w)


def kernel(token_ids, dense_feat, embedding_weight, linear_weight):
  B, S = token_ids.shape
  V, D = embedding_weight.shape
  npad = _SPAD - S
  pad_vals = jnp.arange(B * npad, dtype=jnp.int32).reshape(B, npad) % V
  idx = jnp.concatenate([token_ids.astype(jnp.int32), pad_vals], axis=1)
  idx2 = idx.reshape(_NW, B // (2 * _NW), 2 * _SPAD)
  emb_out = _make_gather(V, D, B, S)(embedding_weight, idx2)
  lin_out = _linear_tc(dense_feat.astype(jnp.float32),
                       linear_weight.astype(jnp.float32))
  return (emb_out, lin_out)


# speedup vs baseline: 26.1053x; 1.7721x over previous
import functools

import jax
import jax.numpy as jnp
from jax import lax
from jax.experimental import pallas as pl
from jax.experimental.pallas import tpu as pltpu
from jax.experimental.pallas import tpu_sc as plsc

_NC = 2
_NS = 16
_NW = _NC * _NS
_CH = 128


@functools.lru_cache(maxsize=None)
def _make_gather(V, D, B):
  b_per_w = B // _NW
  nch = b_per_w // _CH
  mesh = plsc.VectorSubcoreMesh(core_axis_name="c", subcore_axis_name="s")

  @functools.partial(
      pl.kernel,
      mesh=mesh,
      out_type=jax.ShapeDtypeStruct((B, D), jnp.float32),
      scratch_types=[
          pltpu.VMEM((nch, _CH), jnp.int32),
          pltpu.VMEM((_CH, D), jnp.float32),
          pltpu.VMEM((_CH, D), jnp.float32),
          pltpu.SemaphoreType.DMA,
          pltpu.SemaphoreType.DMA,
      ],
      compiler_params=pltpu.CompilerParams(use_tc_tiling_on_sc=True),
  )
  def gather(table_hbm, idx_hbm, out_hbm, idx_v, buf0, buf1, sem0, sem1):
    wid = lax.axis_index("s") * _NC + lax.axis_index("c")
    base = wid * b_per_w
    pltpu.sync_copy(idx_hbm.at[wid], idx_v)

    def g(j, buf, sem):
      return pltpu.make_async_copy(table_hbm.at[idx_v.at[j]], buf, sem)

    g(0, buf0, sem0).start()

    def body(i, carry):
      j0 = 2 * i
      g(j0 + 1, buf1, sem1).start()
      g(j0, buf0, sem0).wait()
      pltpu.sync_copy(buf0, out_hbm.at[pl.ds(base + j0 * _CH, _CH)])

      @pl.when(j0 + 2 < nch)
      def _():
        g(j0 + 2, buf0, sem0).start()

      g(j0 + 1, buf1, sem1).wait()
      pltpu.sync_copy(buf1, out_hbm.at[pl.ds(base + (j0 + 1) * _CH, _CH)])
      return carry

    lax.fori_loop(0, nch // 2, body, 0)

  return gather


def _linear_tc(x, w):
  def mm(x_ref, w_ref, o_ref):
    o_ref[...] = lax.dot_general(
        x_ref[...], w_ref[...], (((1,), (1,)), ((), ())),
        preferred_element_type=jnp.float32)

  return pl.pallas_call(
      mm,
      out_shape=jax.ShapeDtypeStruct((x.shape[0], w.shape[0]), jnp.float32),
  )(x, w)


def kernel(token_ids, dense_feat, embedding_weight, linear_weight):
  B, S = token_ids.shape
  V, D = embedding_weight.shape
  n = B * S
  idx = token_ids.astype(jnp.int32).T.reshape(-1)
  idx3 = idx.reshape(_NW, n // (_NW * _CH), _CH)
  emb_flat = _make_gather(V, D, n)(embedding_weight, idx3)
  emb_out = emb_flat.reshape(S, B, D).transpose(1, 0, 2)
  lin_out = _linear_tc(dense_feat.astype(jnp.float32),
                       linear_weight.astype(jnp.float32))
  return (emb_out, lin_out)


# speedup vs baseline: 26.6768x; 1.0219x over previous
import functools

import jax
import jax.numpy as jnp
from jax import lax
from jax.experimental import pallas as pl
from jax.experimental.pallas import tpu as pltpu
from jax.experimental.pallas import tpu_sc as plsc

_NC = 2
_NS = 16
_NW = _NC * _NS
_CH = 128


@functools.lru_cache(maxsize=None)
def _make_gather(V, D, B):
  b_per_w = B // _NW
  nch = b_per_w // _CH
  mesh = plsc.VectorSubcoreMesh(core_axis_name="c", subcore_axis_name="s")

  nbuf = 5

  @functools.partial(
      pl.kernel,
      mesh=mesh,
      out_type=jax.ShapeDtypeStruct((B, D), jnp.float32),
      scratch_types=[
          pltpu.VMEM((nch, _CH), jnp.int32),
          [pltpu.VMEM((_CH, D), jnp.float32) for _ in range(nbuf)],
          [pltpu.SemaphoreType.DMA for _ in range(nbuf)],
          [pltpu.SemaphoreType.DMA for _ in range(nbuf)],
      ],
      compiler_params=pltpu.CompilerParams(use_tc_tiling_on_sc=True),
  )
  def gather(table_hbm, idx_hbm, out_hbm, idx_v, bufs, gsems, wsems):
    wid = lax.axis_index("s") * _NC + lax.axis_index("c")
    base = wid * b_per_w
    pltpu.sync_copy(idx_hbm.at[wid], idx_v)

    def g(j, b):
      return pltpu.make_async_copy(table_hbm.at[idx_v.at[j]], bufs[b],
                                   gsems[b])

    def wr(j, b):
      return pltpu.make_async_copy(
          bufs[b], out_hbm.at[pl.ds(base + j * _CH, _CH)], wsems[b])

    pf = 2
    for b in range(pf):
      g(b, b).start()

    def body(i, carry):
      for b in range(nbuf):
        j = nbuf * i + b
        g(j, b).wait()
        wr(j, b).start()
        jp = j + pf
        bp = (b + pf) % nbuf

        @pl.when(jnp.logical_and(jp < nch, j >= nbuf - pf))
        def _():
          wr(jp, bp).wait()

        @pl.when(jp < nch)
        def _():
          g(jp, bp).start()
      return carry

    lax.fori_loop(0, nch // nbuf, body, 0)
    for b in range(nbuf):
      wr(0, b).wait()

  return gather


def _linear_tc(x, w):
  def mm(x_ref, w_ref, o_ref):
    o_ref[...] = lax.dot_general(
        x_ref[...], w_ref[...], (((1,), (1,)), ((), ())),
        preferred_element_type=jnp.float32)

  return pl.pallas_call(
      mm,
      out_shape=jax.ShapeDtypeStruct((x.shape[0], w.shape[0]), jnp.float32),
  )(x, w)


def kernel(token_ids, dense_feat, embedding_weight, linear_weight):
  B, S = token_ids.shape
  V, D = embedding_weight.shape
  n = B * S
  idx = token_ids.astype(jnp.int32).T.reshape(-1)
  idx3 = idx.reshape(_NW, n // (_NW * _CH), _CH)
  emb_flat = _make_gather(V, D, n)(embedding_weight, idx3)
  emb_out = emb_flat.reshape(S, B, D).transpose(1, 0, 2)
  lin_out = _linear_tc(dense_feat.astype(jnp.float32),
                       linear_weight.astype(jnp.float32))
  return (emb_out, lin_out)
